# initial kernel scaffold (unmeasured)
import jax
import jax.numpy as jnp
from jax import lax
from jax.experimental import pallas as pl
from jax.experimental.pallas import tpu as pltpu

N_DEV = 8


def kernel(x, w_mat):
    m, kp = x.shape
    _, n = w_mat.shape
    mp = m // N_DEV

    xb = x.astype(jnp.bfloat16)
    wb = w_mat.astype(jnp.bfloat16)

    def body(x_ref, w_ref, out_ref, comm_ref, amax_ref,
             ring_send_sems, ring_recv_sems, amax_send_sems, amax_recv_sem):
        my = lax.axis_index("i")
        left = lax.rem(my + N_DEV - 1, N_DEV)
        right = lax.rem(my + 1, N_DEV)

        barrier_sem = pltpu.get_barrier_semaphore()
        for nbr in (left, right):
            pl.semaphore_signal(barrier_sem, inc=1, device_id=(nbr,),
                                device_id_type=pl.DeviceIdType.MESH)
        pl.semaphore_wait(barrier_sem, 2)

        w = w_ref[...]

        c0 = lax.rem(my + N_DEV - 1, N_DEV)
        comm_ref[0] = jnp.dot(
            x_ref[pl.ds(c0 * mp, mp), :], w,
            preferred_element_type=jnp.float32).astype(jnp.bfloat16)

        for s in range(N_DEV - 1):
            send_slot = s % 2
            recv_slot = (s + 1) % 2
            rdma = pltpu.make_async_remote_copy(
                src_ref=comm_ref.at[send_slot],
                dst_ref=comm_ref.at[recv_slot],
                send_sem=ring_send_sems.at[send_slot],
                recv_sem=ring_recv_sems.at[recv_slot],
                device_id=(right,),
                device_id_type=pl.DeviceIdType.MESH,
            )
            rdma.start()
            c_recv = lax.rem(my + (2 * N_DEV - 2 - s), N_DEV)
            partial = jnp.dot(x_ref[pl.ds(c_recv * mp, mp), :], w,
                              preferred_element_type=jnp.float32)
            rdma.wait()
            acc = comm_ref[recv_slot].astype(jnp.float32) + partial
            if s < N_DEV - 2:
                comm_ref[recv_slot] = acc.astype(jnp.bfloat16)
            else:
                out_ref[...] = acc

        local_amax = jnp.max(jnp.abs(out_ref[...]))
        amax_ref[my] = jnp.broadcast_to(local_amax, (8, 128)).astype(jnp.float32)
        sends = []
        for o in range(1, N_DEV):
            tgt = lax.rem(my + o, N_DEV)
            send = pltpu.make_async_remote_copy(
                src_ref=amax_ref.at[my],
                dst_ref=amax_ref.at[my],
                send_sem=amax_send_sems.at[o - 1],
                recv_sem=amax_recv_sem,
                device_id=(tgt,),
                device_id_type=pl.DeviceIdType.MESH,
            )
            send.start()
            sends.append(send)
        for o in range(1, N_DEV):
            src = lax.rem(my + o, N_DEV)
            recv = pltpu.make_async_remote_copy(
                src_ref=amax_ref.at[src],
                dst_ref=amax_ref.at[src],
                send_sem=amax_send_sems.at[o - 1],
                recv_sem=amax_recv_sem,
                device_id=(src,),
                device_id_type=pl.DeviceIdType.MESH,
            )
            recv.wait_recv()
        for send in sends:
            send.wait_send()

        gmax = jnp.max(amax_ref[...])
        inv_scale = 448.0 / gmax
        scale = gmax / 448.0
        q = jnp.clip(out_ref[...] * inv_scale, -448.0, 448.0)
        out_ref[...] = q.astype(jnp.float8_e4m3fn).astype(jnp.float32) * scale

    return pl.pallas_call(
        body,
        out_shape=jax.ShapeDtypeStruct((mp, n), jnp.float32),
        in_specs=[pl.BlockSpec(memory_space=pltpu.VMEM),
                  pl.BlockSpec(memory_space=pltpu.VMEM)],
        out_specs=pl.BlockSpec(memory_space=pltpu.VMEM),
        scratch_shapes=[
            pltpu.VMEM((2, mp, n), jnp.bfloat16),
            pltpu.VMEM((N_DEV, 8, 128), jnp.float32),
            pltpu.SemaphoreType.DMA((2,)),
            pltpu.SemaphoreType.DMA((2,)),
            pltpu.SemaphoreType.DMA((N_DEV - 1,)),
            pltpu.SemaphoreType.DMA,
        ],
        compiler_params=pltpu.CompilerParams(
            collective_id=0,
            vmem_limit_bytes=128 * 1024 * 1024,
        ),
    )(xb, wb)


# baseline (device time: 730355 ns/iter reference)
import jax
import jax.numpy as jnp
from jax import lax
from jax.experimental import pallas as pl
from jax.experimental.pallas import tpu as pltpu

N_DEV = 8
TILE_N = 2048


def kernel(x, w_mat):
    m, kp = x.shape
    _, n = w_mat.shape
    mp = m // N_DEV
    nt = n // TILE_N

    xb = x.astype(jnp.bfloat16)
    wb = w_mat.astype(jnp.bfloat16)

    def body(x_ref, w_ref, out_ref, comm_ref, amax_ref,
             ring_send_sems, ring_recv_sems, amax_send_sems, amax_recv_sem):
        my = lax.axis_index("i")
        left = lax.rem(my + N_DEV - 1, N_DEV)
        right = lax.rem(my + 1, N_DEV)

        barrier_sem = pltpu.get_barrier_semaphore()
        for nbr in (left, right):
            pl.semaphore_signal(barrier_sem, inc=1, device_id=(nbr,),
                                device_id_type=pl.DeviceIdType.MESH)
        pl.semaphore_wait(barrier_sem, 2)

        def mm_tile(c, t):
            return jnp.dot(
                x_ref[pl.ds(c * mp, mp), :],
                w_ref[:, pl.ds(t * TILE_N, TILE_N)],
                preferred_element_type=jnp.float32)

        c0 = lax.rem(my + N_DEV - 1, N_DEV)
        for t in range(nt):
            comm_ref[0, :, pl.ds(t * TILE_N, TILE_N)] = (
                mm_tile(c0, t).astype(jnp.bfloat16))

        for s in range(N_DEV - 1):
            send_slot = s % 2
            recv_slot = (s + 1) % 2
            rdma = pltpu.make_async_remote_copy(
                src_ref=comm_ref.at[send_slot],
                dst_ref=comm_ref.at[recv_slot],
                send_sem=ring_send_sems.at[send_slot],
                recv_sem=ring_recv_sems.at[recv_slot],
                device_id=(right,),
                device_id_type=pl.DeviceIdType.MESH,
            )
            rdma.start()
            rdma.wait()
            c_recv = lax.rem(my + (2 * N_DEV - 2 - s), N_DEV)
            for t in range(nt):
                sl = pl.ds(t * TILE_N, TILE_N)
                acc = comm_ref[recv_slot, :, sl].astype(jnp.float32) + mm_tile(c_recv, t)
                if s < N_DEV - 2:
                    comm_ref[recv_slot, :, sl] = acc.astype(jnp.bfloat16)
                else:
                    out_ref[:, sl] = acc

        local_amax = jnp.float32(0.0)
        for t in range(nt):
            sl = pl.ds(t * TILE_N, TILE_N)
            local_amax = jnp.maximum(local_amax, jnp.max(jnp.abs(out_ref[:, sl])))
        amax_ref[my] = jnp.broadcast_to(local_amax, (8, 128)).astype(jnp.float32)
        sends = []
        for o in range(1, N_DEV):
            tgt = lax.rem(my + o, N_DEV)
            send = pltpu.make_async_remote_copy(
                src_ref=amax_ref.at[my],
                dst_ref=amax_ref.at[my],
                send_sem=amax_send_sems.at[o - 1],
                recv_sem=amax_recv_sem,
                device_id=(tgt,),
                device_id_type=pl.DeviceIdType.MESH,
            )
            send.start()
            sends.append(send)
        for o in range(1, N_DEV):
            src = lax.rem(my + o, N_DEV)
            recv = pltpu.make_async_remote_copy(
                src_ref=amax_ref.at[src],
                dst_ref=amax_ref.at[src],
                send_sem=amax_send_sems.at[o - 1],
                recv_sem=amax_recv_sem,
                device_id=(src,),
                device_id_type=pl.DeviceIdType.MESH,
            )
            recv.wait_recv()
        for send in sends:
            send.wait_send()

        gmax = jnp.max(amax_ref[...])
        inv_scale = 448.0 / gmax
        scale = gmax / 448.0
        for t in range(nt):
            sl = pl.ds(t * TILE_N, TILE_N)
            q = jnp.clip(out_ref[:, sl] * inv_scale, -448.0, 448.0)
            out_ref[:, sl] = q.astype(jnp.float8_e4m3fn).astype(jnp.float32) * scale

    return pl.pallas_call(
        body,
        out_shape=jax.ShapeDtypeStruct((mp, n), jnp.float32),
        in_specs=[pl.BlockSpec(memory_space=pltpu.VMEM),
                  pl.BlockSpec(memory_space=pltpu.VMEM)],
        out_specs=pl.BlockSpec(memory_space=pltpu.VMEM),
        scratch_shapes=[
            pltpu.VMEM((2, mp, n), jnp.bfloat16),
            pltpu.VMEM((N_DEV, 8, 128), jnp.float32),
            pltpu.SemaphoreType.DMA((2,)),
            pltpu.SemaphoreType.DMA((2,)),
            pltpu.SemaphoreType.DMA((N_DEV - 1,)),
            pltpu.SemaphoreType.DMA,
        ],
        compiler_params=pltpu.CompilerParams(
            collective_id=0,
            vmem_limit_bytes=63 * 1024 * 1024,
        ),
    )(xb, wb)


# device time: 367734 ns/iter; 1.9861x vs baseline; 1.9861x over previous
import jax
import jax.numpy as jnp
from jax import lax
from jax.experimental import pallas as pl
from jax.experimental.pallas import tpu as pltpu

N_DEV = 8
N_STREAM = 4
TILE_N = 2048


def kernel(x, w_mat):
    m, kp = x.shape
    _, n = w_mat.shape
    mp = m // N_DEV

    xb = x.astype(jnp.bfloat16)
    wb = w_mat.astype(jnp.bfloat16)

    def body(x_ref, w_ref, out_ref, comm_ref, amax_ref,
             send_sems, recv_sems, amax_send_sems, amax_recv_sem):
        my = lax.axis_index("i")
        left = lax.rem(my + N_DEV - 1, N_DEV)
        right = lax.rem(my + 1, N_DEV)

        barrier_sem = pltpu.get_barrier_semaphore()
        for nbr in (left, right):
            pl.semaphore_signal(barrier_sem, inc=1, device_id=(nbr,),
                                device_id_type=pl.DeviceIdType.MESH)
        pl.semaphore_wait(barrier_sem, 2)

        streams = []
        for q in range(N_STREAM):
            dir_r = (q % 2 == 0)
            g = q // 2
            col0 = (0 if dir_r else n // 2) + g * TILE_N
            streams.append((q, dir_r, col0))

        def mm(c, col0):
            return jnp.dot(
                x_ref[pl.ds(c * mp, mp), :],
                w_ref[:, col0:col0 + TILE_N],
                preferred_element_type=jnp.float32)

        def send_chunk_idx(dir_r, s):
            off = (N_DEV - 1 - s) if dir_r else (N_DEV + 1 + s)
            return lax.rem(my + off, N_DEV)

        def recv_chunk_idx(dir_r, s):
            off = (2 * N_DEV - 2 - s) if dir_r else (N_DEV + 2 + s)
            return lax.rem(my + off, N_DEV)

        def make_rdma(q, dir_r, s):
            return pltpu.make_async_remote_copy(
                src_ref=comm_ref.at[q, s % 2],
                dst_ref=comm_ref.at[q, (s + 1) % 2],
                send_sem=send_sems.at[q, s % 2],
                recv_sem=recv_sems.at[q, (s + 1) % 2],
                device_id=(right if dir_r else left,),
                device_id_type=pl.DeviceIdType.MESH,
            )

        descs = {}
        for q, dir_r, col0 in streams:
            comm_ref[q, 0] = mm(send_chunk_idx(dir_r, 0), col0).astype(jnp.bfloat16)
            d = make_rdma(q, dir_r, 0)
            d.start()
            descs[(q, 0)] = d

        amax_local = jnp.float32(0.0)
        for s in range(N_DEV - 1):
            recv_slot = (s + 1) % 2
            for q, dir_r, col0 in streams:
                descs[(q, s)].wait_recv()
                part = mm(recv_chunk_idx(dir_r, s), col0)
                acc = comm_ref[q, recv_slot].astype(jnp.float32) + part
                if s < N_DEV - 2:
                    comm_ref[q, recv_slot] = acc.astype(jnp.bfloat16)
                    if s >= 1:
                        descs[(q, s - 1)].wait_send()
                    d = make_rdma(q, dir_r, s + 1)
                    d.start()
                    descs[(q, s + 1)] = d
                else:
                    out_ref[:, col0:col0 + TILE_N] = acc
                    amax_local = jnp.maximum(amax_local, jnp.max(jnp.abs(acc)))
        for q, dir_r, col0 in streams:
            descs[(q, N_DEV - 3)].wait_send()
            descs[(q, N_DEV - 2)].wait_send()

        amax_ref[my] = jnp.broadcast_to(amax_local, (8, 128)).astype(jnp.float32)
        sends = []
        for o in range(1, N_DEV):
            tgt = lax.rem(my + o, N_DEV)
            send = pltpu.make_async_remote_copy(
                src_ref=amax_ref.at[my],
                dst_ref=amax_ref.at[my],
                send_sem=amax_send_sems.at[o - 1],
                recv_sem=amax_recv_sem,
                device_id=(tgt,),
                device_id_type=pl.DeviceIdType.MESH,
            )
            send.start()
            sends.append(send)
        for o in range(1, N_DEV):
            src = lax.rem(my + o, N_DEV)
            recv = pltpu.make_async_remote_copy(
                src_ref=amax_ref.at[src],
                dst_ref=amax_ref.at[src],
                send_sem=amax_send_sems.at[o - 1],
                recv_sem=amax_recv_sem,
                device_id=(src,),
                device_id_type=pl.DeviceIdType.MESH,
            )
            recv.wait_recv()
        for send in sends:
            send.wait_send()

        gmax = jnp.max(amax_ref[...])
        inv_scale = 448.0 / gmax
        scale = gmax / 448.0
        for t in range(n // TILE_N):
            sl = pl.ds(t * TILE_N, TILE_N)
            q8 = jnp.clip(out_ref[:, sl] * inv_scale, -448.0, 448.0)
            out_ref[:, sl] = q8.astype(jnp.float8_e4m3fn).astype(jnp.float32) * scale

    return pl.pallas_call(
        body,
        out_shape=jax.ShapeDtypeStruct((mp, n), jnp.float32),
        in_specs=[pl.BlockSpec(memory_space=pltpu.VMEM),
                  pl.BlockSpec(memory_space=pltpu.VMEM)],
        out_specs=pl.BlockSpec(memory_space=pltpu.VMEM),
        scratch_shapes=[
            pltpu.VMEM((N_STREAM, 2, mp, TILE_N), jnp.bfloat16),
            pltpu.VMEM((N_DEV, 8, 128), jnp.float32),
            pltpu.SemaphoreType.DMA((N_STREAM, 2)),
            pltpu.SemaphoreType.DMA((N_STREAM, 2)),
            pltpu.SemaphoreType.DMA((N_DEV - 1,)),
            pltpu.SemaphoreType.DMA,
        ],
        compiler_params=pltpu.CompilerParams(
            collective_id=0,
            vmem_limit_bytes=63 * 1024 * 1024,
        ),
    )(xb, wb)


# device time: 366508 ns/iter; 1.9927x vs baseline; 1.0033x over previous
import jax
import jax.numpy as jnp
from jax import lax
from jax.experimental import pallas as pl
from jax.experimental.pallas import tpu as pltpu

N_DEV = 8
N_STREAM = 8
TILE_N = 1024


def kernel(x, w_mat):
    m, kp = x.shape
    _, n = w_mat.shape
    mp = m // N_DEV

    xb = x.astype(jnp.bfloat16)
    wb = w_mat.astype(jnp.bfloat16)

    def body(x_ref, w_ref, out_ref, comm_ref, amax_ref,
             send_sems, recv_sems, amax_send_sems, amax_recv_sem):
        my = lax.axis_index("i")
        left = lax.rem(my + N_DEV - 1, N_DEV)
        right = lax.rem(my + 1, N_DEV)

        barrier_sem = pltpu.get_barrier_semaphore()
        for nbr in (left, right):
            pl.semaphore_signal(barrier_sem, inc=1, device_id=(nbr,),
                                device_id_type=pl.DeviceIdType.MESH)
        pl.semaphore_wait(barrier_sem, 2)

        streams = []
        for q in range(N_STREAM):
            dir_r = (q % 2 == 0)
            g = q // 2
            col0 = (0 if dir_r else n // 2) + g * TILE_N
            streams.append((q, dir_r, col0))

        def mm(c, col0):
            return jnp.dot(
                x_ref[pl.ds(c * mp, mp), :],
                w_ref[:, col0:col0 + TILE_N],
                preferred_element_type=jnp.float32)

        def send_chunk_idx(dir_r, s):
            off = (N_DEV - 1 - s) if dir_r else (N_DEV + 1 + s)
            return lax.rem(my + off, N_DEV)

        def recv_chunk_idx(dir_r, s):
            off = (2 * N_DEV - 2 - s) if dir_r else (N_DEV + 2 + s)
            return lax.rem(my + off, N_DEV)

        def make_rdma(q, dir_r, s):
            return pltpu.make_async_remote_copy(
                src_ref=comm_ref.at[q, s % 2],
                dst_ref=comm_ref.at[q, (s + 1) % 2],
                send_sem=send_sems.at[q, s % 2],
                recv_sem=recv_sems.at[q, (s + 1) % 2],
                device_id=(right if dir_r else left,),
                device_id_type=pl.DeviceIdType.MESH,
            )

        descs = {}
        for q, dir_r, col0 in streams:
            comm_ref[q, 0] = mm(send_chunk_idx(dir_r, 0), col0).astype(jnp.bfloat16)
            d = make_rdma(q, dir_r, 0)
            d.start()
            descs[(q, 0)] = d

        amax_local = jnp.float32(0.0)
        for s in range(N_DEV - 1):
            recv_slot = (s + 1) % 2
            for q, dir_r, col0 in streams:
                part = mm(recv_chunk_idx(dir_r, s), col0)
                descs[(q, s)].wait_recv()
                acc = comm_ref[q, recv_slot].astype(jnp.float32) + part
                if s < N_DEV - 2:
                    comm_ref[q, recv_slot] = acc.astype(jnp.bfloat16)
                    if s >= 1:
                        descs[(q, s - 1)].wait_send()
                    d = make_rdma(q, dir_r, s + 1)
                    d.start()
                    descs[(q, s + 1)] = d
                else:
                    out_ref[:, col0:col0 + TILE_N] = acc
                    amax_local = jnp.maximum(amax_local, jnp.max(jnp.abs(acc)))
        for q, dir_r, col0 in streams:
            descs[(q, N_DEV - 3)].wait_send()
            descs[(q, N_DEV - 2)].wait_send()

        amax_ref[my] = jnp.broadcast_to(amax_local, (8, 128)).astype(jnp.float32)
        sends = []
        for o in range(1, N_DEV):
            tgt = lax.rem(my + o, N_DEV)
            send = pltpu.make_async_remote_copy(
                src_ref=amax_ref.at[my],
                dst_ref=amax_ref.at[my],
                send_sem=amax_send_sems.at[o - 1],
                recv_sem=amax_recv_sem,
                device_id=(tgt,),
                device_id_type=pl.DeviceIdType.MESH,
            )
            send.start()
            sends.append(send)
        for o in range(1, N_DEV):
            src = lax.rem(my + o, N_DEV)
            recv = pltpu.make_async_remote_copy(
                src_ref=amax_ref.at[src],
                dst_ref=amax_ref.at[src],
                send_sem=amax_send_sems.at[o - 1],
                recv_sem=amax_recv_sem,
                device_id=(src,),
                device_id_type=pl.DeviceIdType.MESH,
            )
            recv.wait_recv()
        for send in sends:
            send.wait_send()

        gmax = jnp.max(amax_ref[...])
        inv_scale = 448.0 / gmax
        scale = gmax / 448.0
        for t in range(n // TILE_N):
            sl = pl.ds(t * TILE_N, TILE_N)
            q8 = jnp.clip(out_ref[:, sl] * inv_scale, -448.0, 448.0)
            out_ref[:, sl] = q8.astype(jnp.float8_e4m3fn).astype(jnp.float32) * scale

    return pl.pallas_call(
        body,
        out_shape=jax.ShapeDtypeStruct((mp, n), jnp.float32),
        in_specs=[pl.BlockSpec(memory_space=pltpu.VMEM),
                  pl.BlockSpec(memory_space=pltpu.VMEM)],
        out_specs=pl.BlockSpec(memory_space=pltpu.VMEM),
        scratch_shapes=[
            pltpu.VMEM((N_STREAM, 2, mp, TILE_N), jnp.bfloat16),
            pltpu.VMEM((N_DEV, 8, 128), jnp.float32),
            pltpu.SemaphoreType.DMA((N_STREAM, 2)),
            pltpu.SemaphoreType.DMA((N_STREAM, 2)),
            pltpu.SemaphoreType.DMA((N_DEV - 1,)),
            pltpu.SemaphoreType.DMA,
        ],
        compiler_params=pltpu.CompilerParams(
            collective_id=0,
            vmem_limit_bytes=63 * 1024 * 1024,
        ),
    )(xb, wb)
